# baseline (speedup 1.0000x reference)
import jax
import jax.numpy as jnp
from jax import lax
from jax.experimental import pallas as pl
from jax.experimental.pallas import tpu as pltpu
from jax.experimental.pallas import tpu_sc as plsc

EMB_DIM = 128
BPW = 128
NBUF = 6
LEAD = 4

_vector_mesh = plsc.VectorSubcoreMesh(
    core_axis_name="core", subcore_axis_name="subcore"
)
_NW = 32


def _gather_rows(table, xt):
    hist, batch = xt.shape
    nchunk = hist

    @pl.kernel(
        out_type=jax.ShapeDtypeStruct((hist, batch, EMB_DIM), table.dtype),
        mesh=_vector_mesh,
        scratch_types=[
            pltpu.VMEM((hist, BPW), jnp.int32),
            pltpu.VMEM((NBUF, BPW, EMB_DIM), table.dtype),
            pltpu.SemaphoreType.DMA((NBUF,)),
            pltpu.SemaphoreType.DMA((NBUF,)),
        ],
    )
    def _kernel(table_hbm, xt_hbm, out_hbm, idx_v, bufs, gsem, osem):
        wid = lax.axis_index("subcore") * 2 + lax.axis_index("core")
        col_base = wid * BPW

        def g_start(k, b):
            for h in range(2):
                pltpu.make_async_copy(
                    table_hbm.at[idx_v.at[k].at[pl.ds(h * 64, 64)]],
                    bufs.at[b].at[pl.ds(h * 64, 64)],
                    gsem.at[b],
                ).start()

        def g_wait(b):
            for h in range(2):
                pltpu.make_async_copy(
                    table_hbm.at[idx_v.at[0].at[pl.ds(h * 64, 64)]],
                    bufs.at[b].at[pl.ds(h * 64, 64)],
                    gsem.at[b],
                ).wait()

        def o_start(k, b):
            pltpu.make_async_copy(
                bufs.at[b],
                out_hbm.at[k].at[pl.ds(col_base, BPW)],
                osem.at[b],
            ).start()

        def o_wait(b):
            pltpu.make_async_copy(
                bufs.at[b],
                out_hbm.at[0].at[pl.ds(0, BPW)],
                osem.at[b],
            ).wait()

        pltpu.sync_copy(xt_hbm.at[:, pl.ds(col_base, BPW)], idx_v)

        def step(k, kmod):
            if k + LEAD < nchunk:
                if k >= 2:
                    o_wait((kmod + LEAD) % NBUF)
                g_start(k + LEAD, (kmod + LEAD) % NBUF)
            g_wait(kmod % NBUF)
            o_start(k, kmod % NBUF)

        for b in range(LEAD):
            g_start(b, b)

        step(0, 0)
        step(1, 1)

        ngroups = (nchunk - LEAD - 2) // NBUF

        @pl.loop(0, ngroups)
        def _(g):
            for b in range(NBUF):
                k = g * NBUF + b + 2
                o_wait((b + 2 + LEAD) % NBUF)
                g_start(k + LEAD, (b + 2 + LEAD) % NBUF)
                g_wait((b + 2) % NBUF)
                o_start(k, (b + 2) % NBUF)

        for k in range(ngroups * NBUF + 2, nchunk):
            step(k, k % NBUF)

        for b in range(NBUF):
            o_wait(b)

    return _kernel(table, xt)


def kernel(x, emb_weight):
    batch, hist = x.shape
    xt = x.T.astype(jnp.int32)
    out3 = _gather_rows(emb_weight, xt)
    return jnp.transpose(out3, (1, 0, 2))


# speedup vs baseline: 1.0000x; 1.0000x over previous
import jax
import jax.numpy as jnp
from jax import lax
from jax.experimental import pallas as pl
from jax.experimental.pallas import tpu as pltpu
from jax.experimental.pallas import tpu_sc as plsc

EMB_DIM = 128
BPW = 128
NBUF = 5
LEAD = 3

_vector_mesh = plsc.VectorSubcoreMesh(
    core_axis_name="core", subcore_axis_name="subcore"
)
_NW = 32


def _gather_rows(table, xt):
    hist, batch = xt.shape
    nchunk = hist

    @pl.kernel(
        out_type=jax.ShapeDtypeStruct((hist, batch, EMB_DIM), table.dtype),
        mesh=_vector_mesh,
        scratch_types=[
            pltpu.VMEM((hist, BPW), jnp.int32),
            pltpu.VMEM((NBUF, BPW, EMB_DIM), table.dtype),
            pltpu.SemaphoreType.DMA((NBUF,)),
            pltpu.SemaphoreType.DMA((NBUF,)),
        ],
    )
    def _kernel(table_hbm, xt_hbm, out_hbm, idx_v, bufs, gsem, osem):
        wid = lax.axis_index("subcore") * 2 + lax.axis_index("core")
        col_base = wid * BPW

        def g_start(k, b):
            pltpu.make_async_copy(
                table_hbm.at[idx_v.at[k]], bufs.at[b], gsem.at[b]
            ).start()

        def g_wait(b):
            pltpu.make_async_copy(
                table_hbm.at[idx_v.at[0]], bufs.at[b], gsem.at[b]
            ).wait()

        def o_start(k, b):
            pltpu.make_async_copy(
                bufs.at[b],
                out_hbm.at[k].at[pl.ds(col_base, BPW)],
                osem.at[b],
            ).start()

        def o_wait(b):
            pltpu.make_async_copy(
                bufs.at[b],
                out_hbm.at[0].at[pl.ds(0, BPW)],
                osem.at[b],
            ).wait()

        pltpu.sync_copy(xt_hbm.at[:, pl.ds(col_base, BPW)], idx_v)

        def step(k, kmod):
            if k + LEAD < nchunk:
                if k >= 2:
                    o_wait((kmod + LEAD) % NBUF)
                g_start(k + LEAD, (kmod + LEAD) % NBUF)
            g_wait(kmod % NBUF)
            o_start(k, kmod % NBUF)

        for b in range(LEAD):
            g_start(b, b)

        step(0, 0)
        step(1, 1)

        ngroups = (nchunk - LEAD - 2) // NBUF

        @pl.loop(0, ngroups)
        def _(g):
            for b in range(NBUF):
                k = g * NBUF + b + 2
                o_wait((b + 2 + LEAD) % NBUF)
                g_start(k + LEAD, (b + 2 + LEAD) % NBUF)
                g_wait((b + 2) % NBUF)
                o_start(k, (b + 2) % NBUF)

        for k in range(ngroups * NBUF + 2, nchunk):
            step(k, k % NBUF)

        for b in range(NBUF):
            o_wait(b)

    return _kernel(table, xt)


def kernel(x, emb_weight):
    batch, hist = x.shape
    xt = x.T.astype(jnp.int32)
    out3 = _gather_rows(emb_weight, xt)
    return jnp.transpose(out3, (1, 0, 2))
